# baseline (speedup 1.0000x reference)
import jax
import jax.numpy as jnp
from jax import lax
from jax.experimental import pallas as pl
from jax.experimental.pallas import tpu as pltpu
from jax.experimental.pallas import tpu_sc as plsc

_S = 64
_S3 = _S ** 3
_WREG = 25.0
_B = 16
_N = 16384
_P = 3
_NC = 2
_NS = 16
_NW = _NC * _NS
_HP = 96
_HPN = (_P * _B * _N) // _HP
_C = 2048
_K = _C // 128
_G = _C // 16
_CHUNKS = 12


def _vsqrt(s, bitb):
  bitb[0] = s
  i = bitb.bitcast(jnp.int32)[0]
  bitb.bitcast(jnp.int32)[0] = jnp.int32(0x5F3759DF) - (i >> 1)
  y = bitb[0]
  y = y * (1.5 - 0.5 * s * y * y)
  y = y * (1.5 - 0.5 * s * y * y)
  y = y * (1.5 - 0.5 * s * y * y)
  return s * y


def _sc_body(pts, params, regp, aux, vox, out,
             xb, yb, zb, rxb, ryb, rzb, ixb, iyb, izb, ivb,
             txb, tyb, tzb, voxb, pbuf, rbuf, accb, bitb, sem):
  wid = lax.axis_index("s") * _NC + lax.axis_index("c")

  def chunk(u, acc):
    t = u // 4
    c = u % 4
    h = wid + _NW * t
    pair = h // 2
    half = h % 2
    b = pair % _B
    n0 = half * _HPN + c * _C
    pltpu.sync_copy(params.at[pair], pbuf)
    nx = pbuf[0]
    ny = pbuf[1]
    nz = pbuf[2]
    off = pbuf[3]
    s2 = 2.0 / (nx * nx + ny * ny + nz * nz)
    pltpu.sync_copy(pts.at[0, b, pl.ds(n0, _C)], xb)
    pltpu.sync_copy(pts.at[1, b, pl.ds(n0, _C)], yb)
    pltpu.sync_copy(pts.at[2, b, pl.ds(n0, _C)], zb)
    rowbase = b * _S3

    def grp(g, _):
      o = pl.multiple_of(g * 16, 16)
      x = xb[pl.ds(o, 16)]
      y = yb[pl.ds(o, 16)]
      z = zb[pl.ds(o, 16)]
      d = (x * nx + y * ny + z * nz + off) * s2
      rx = x - d * nx
      ry = y - d * ny
      rz = z - d * nz
      rxb[pl.ds(o, 16)] = rx
      ryb[pl.ds(o, 16)] = ry
      rzb[pl.ds(o, 16)] = rz
      fx = jnp.clip((rx + 0.5) * 64.0, -1.0, 64.0)
      fy = jnp.clip((ry + 0.5) * 64.0, -1.0, 64.0)
      fz = jnp.clip((rz + 0.5) * 64.0, -1.0, 64.0)
      ix = jnp.clip(fx.astype(jnp.int32), 0, _S - 1)
      iy = jnp.clip(fy.astype(jnp.int32), 0, _S - 1)
      iz = jnp.clip(fz.astype(jnp.int32), 0, _S - 1)
      gi = (ix * _S + iy) * _S + iz + rowbase
      gi3 = gi * 3
      j = g // 8
      r0 = (g % 8) * 16
      ixb[j, pl.ds(r0, 16)] = gi3
      iyb[j, pl.ds(r0, 16)] = gi3 + 1
      izb[j, pl.ds(r0, 16)] = gi3 + 2
      ivb[j, pl.ds(r0, 16)] = gi
      return 0

    lax.fori_loop(0, _G, grp, 0)

    def gath(j, _):
      pltpu.async_copy(aux.at[ixb.at[j]], txb.at[j], sem).wait()
      pltpu.async_copy(aux.at[iyb.at[j]], tyb.at[j], sem).wait()
      pltpu.async_copy(aux.at[izb.at[j]], tzb.at[j], sem).wait()
      pltpu.async_copy(vox.at[ivb.at[j]], voxb.at[j], sem).wait()
      return 0

    lax.fori_loop(0, _K, gath, 0)

    def dst(g, a):
      j = g // 8
      r0 = (g % 8) * 16
      o = pl.multiple_of(g * 16, 16)
      vv = voxb[j, pl.ds(r0, 16)]
      dx = rxb[pl.ds(o, 16)] - txb[j, pl.ds(r0, 16)] + 1e-6
      dy = ryb[pl.ds(o, 16)] - tyb[j, pl.ds(r0, 16)] + 1e-6
      dz = rzb[pl.ds(o, 16)] - tzb[j, pl.ds(r0, 16)] + 1e-6
      return a + _vsqrt(dx * dx + dy * dy + dz * dz, bitb) * (1.0 - vv)

    return lax.fori_loop(0, _G, dst, acc)

  acc = lax.fori_loop(0, _CHUNKS, chunk, jnp.zeros((16,), jnp.float32))

  @pl.when(wid == 0)
  def _():
    pltpu.sync_copy(regp, rbuf)
    n = [[rbuf[i, j] for j in range(3)] for i in range(3)]
    linv = []
    for i in range(3):
      li = n[i][0] * n[i][0] + n[i][1] * n[i][1] + n[i][2] * n[i][2]
      linv.append(1.0 / jnp.maximum(li, 1e-24))
    reg = jnp.zeros((16,), jnp.float32)
    for i in range(3):
      di = n[i][i] * n[i][i] * linv[i] - 1.0
      reg = reg + di * di
      for j in range(i + 1, 3):
        q = n[i][j] * n[j][i]
        reg = reg + 2.0 * q * q * (linv[i] * linv[j])
    accb[...] = (acc + _WREG * reg) * (1.0 / _B)

  @pl.when(wid != 0)
  def _():
    accb[...] = acc * (1.0 / _B)

  pltpu.sync_copy(accb, out.at[wid])


@jax.jit
def _sc_loss(pts, params, regp, aux, vox):
  mesh = plsc.VectorSubcoreMesh(core_axis_name="c", subcore_axis_name="s")
  f = pl.kernel(
      _sc_body,
      out_type=jax.ShapeDtypeStruct((_NW, 16), jnp.float32),
      mesh=mesh,
      scratch_types=[
          pltpu.VMEM((_C,), jnp.float32),
          pltpu.VMEM((_C,), jnp.float32),
          pltpu.VMEM((_C,), jnp.float32),
          pltpu.VMEM((_C,), jnp.float32),
          pltpu.VMEM((_C,), jnp.float32),
          pltpu.VMEM((_C,), jnp.float32),
          pltpu.VMEM((_K, 128), jnp.int32),
          pltpu.VMEM((_K, 128), jnp.int32),
          pltpu.VMEM((_K, 128), jnp.int32),
          pltpu.VMEM((_K, 128), jnp.int32),
          pltpu.VMEM((_K, 128), jnp.float32),
          pltpu.VMEM((_K, 128), jnp.float32),
          pltpu.VMEM((_K, 128), jnp.float32),
          pltpu.VMEM((_K, 128), jnp.float32),
          pltpu.VMEM((4, 16), jnp.float32),
          pltpu.VMEM((3, 4, 16), jnp.float32),
          pltpu.VMEM((16,), jnp.float32),
          pltpu.VMEM((1, 16), jnp.float32),
          pltpu.SemaphoreType.DMA,
      ],
  )
  return f(pts, params, regp, aux, vox)


def kernel(point_cloud, auxiliary_data, voxel_data, predicted_planes):
  pts = jnp.transpose(point_cloud, (2, 0, 1))
  params = jnp.broadcast_to(
      predicted_planes.reshape(_P * _B, 4)[:, :, None], (_P * _B, 4, 16))
  regp = jnp.transpose(predicted_planes, (0, 2, 1))
  aux = auxiliary_data.reshape(_B * _S3 * 3)
  vox = voxel_data.reshape(_B * _S3)
  out = _sc_loss(pts, params, regp, aux, vox)
  return jnp.sum(out)


# speedup vs baseline: 1.0201x; 1.0201x over previous
import jax
import jax.numpy as jnp
from jax import lax
from jax.experimental import pallas as pl
from jax.experimental.pallas import tpu as pltpu
from jax.experimental.pallas import tpu_sc as plsc

_S = 64
_S3 = _S ** 3
_WREG = 25.0
_B = 16
_N = 16384
_P = 3
_NC = 2
_NS = 16
_NW = _NC * _NS
_HP = 96
_HPN = (_P * _B * _N) // _HP
_C = 2048
_K = _C // 128
_G = _C // 16
_CHUNKS = 12


def _vsqrt(s, bitb):
  bitb[0] = s
  i = bitb.bitcast(jnp.int32)[0]
  bitb.bitcast(jnp.int32)[0] = jnp.int32(0x5F3759DF) - (i >> 1)
  y = bitb[0]
  y = y * (1.5 - 0.5 * s * y * y)
  y = y * (1.5 - 0.5 * s * y * y)
  y = y * (1.5 - 0.5 * s * y * y)
  return s * y


def _sc_body(pts, params, regp, aux, vox, out,
             xb, yb, zb, rxb, ryb, rzb, ixb, iyb, izb, ivb,
             txb, tyb, tzb, voxb, pbuf, rbuf, accb, bitb, sem):
  wid = lax.axis_index("s") * _NC + lax.axis_index("c")

  def chunk(u, acc):
    t = u // 4
    c = u % 4
    h = wid + _NW * t
    pair = h // 2
    half = h % 2
    b = pair % _B
    n0 = half * _HPN + c * _C
    pltpu.sync_copy(params.at[pair], pbuf)
    nx = pbuf[0]
    ny = pbuf[1]
    nz = pbuf[2]
    off = pbuf[3]
    s2 = 2.0 / (nx * nx + ny * ny + nz * nz)
    pltpu.sync_copy(pts.at[0, b, pl.ds(n0, _C)], xb)
    pltpu.sync_copy(pts.at[1, b, pl.ds(n0, _C)], yb)
    pltpu.sync_copy(pts.at[2, b, pl.ds(n0, _C)], zb)
    rowbase = b * _S3

    def grp(g, _):
      o = pl.multiple_of(g * 16, 16)
      x = xb[pl.ds(o, 16)]
      y = yb[pl.ds(o, 16)]
      z = zb[pl.ds(o, 16)]
      d = (x * nx + y * ny + z * nz + off) * s2
      rx = x - d * nx
      ry = y - d * ny
      rz = z - d * nz
      rxb[pl.ds(o, 16)] = rx
      ryb[pl.ds(o, 16)] = ry
      rzb[pl.ds(o, 16)] = rz
      fx = jnp.clip((rx + 0.5) * 64.0, -1.0, 64.0)
      fy = jnp.clip((ry + 0.5) * 64.0, -1.0, 64.0)
      fz = jnp.clip((rz + 0.5) * 64.0, -1.0, 64.0)
      ix = jnp.clip(fx.astype(jnp.int32), 0, _S - 1)
      iy = jnp.clip(fy.astype(jnp.int32), 0, _S - 1)
      iz = jnp.clip(fz.astype(jnp.int32), 0, _S - 1)
      gi = (ix * _S + iy) * _S + iz + rowbase
      gi3 = gi * 3
      ixb[pl.ds(o, 16)] = gi3
      iyb[pl.ds(o, 16)] = gi3 + 1
      izb[pl.ds(o, 16)] = gi3 + 2
      ivb[pl.ds(o, 16)] = gi
      return 0

    lax.fori_loop(0, _G, grp, 0)

    cx = pltpu.async_copy(aux.at[ixb], txb, sem)
    cy = pltpu.async_copy(aux.at[iyb], tyb, sem)
    cz = pltpu.async_copy(aux.at[izb], tzb, sem)
    cv = pltpu.async_copy(vox.at[ivb], voxb, sem)
    cx.wait()
    cy.wait()
    cz.wait()
    cv.wait()

    def dst(g, a):
      o = pl.multiple_of(g * 16, 16)
      vv = voxb[pl.ds(o, 16)]
      dx = rxb[pl.ds(o, 16)] - txb[pl.ds(o, 16)] + 1e-6
      dy = ryb[pl.ds(o, 16)] - tyb[pl.ds(o, 16)] + 1e-6
      dz = rzb[pl.ds(o, 16)] - tzb[pl.ds(o, 16)] + 1e-6
      return a + _vsqrt(dx * dx + dy * dy + dz * dz, bitb) * (1.0 - vv)

    return lax.fori_loop(0, _G, dst, acc)

  acc = lax.fori_loop(0, _CHUNKS, chunk, jnp.zeros((16,), jnp.float32))

  @pl.when(wid == 0)
  def _():
    pltpu.sync_copy(regp, rbuf)
    n = [[rbuf[i, j] for j in range(3)] for i in range(3)]
    linv = []
    for i in range(3):
      li = n[i][0] * n[i][0] + n[i][1] * n[i][1] + n[i][2] * n[i][2]
      linv.append(1.0 / jnp.maximum(li, 1e-24))
    reg = jnp.zeros((16,), jnp.float32)
    for i in range(3):
      di = n[i][i] * n[i][i] * linv[i] - 1.0
      reg = reg + di * di
      for j in range(i + 1, 3):
        q = n[i][j] * n[j][i]
        reg = reg + 2.0 * q * q * (linv[i] * linv[j])
    accb[...] = (acc + _WREG * reg) * (1.0 / _B)

  @pl.when(wid != 0)
  def _():
    accb[...] = acc * (1.0 / _B)

  pltpu.sync_copy(accb, out.at[wid])


@jax.jit
def _sc_loss(pts, params, regp, aux, vox):
  mesh = plsc.VectorSubcoreMesh(core_axis_name="c", subcore_axis_name="s")
  f = pl.kernel(
      _sc_body,
      out_type=jax.ShapeDtypeStruct((_NW, 16), jnp.float32),
      mesh=mesh,
      scratch_types=[
          pltpu.VMEM((_C,), jnp.float32),
          pltpu.VMEM((_C,), jnp.float32),
          pltpu.VMEM((_C,), jnp.float32),
          pltpu.VMEM((_C,), jnp.float32),
          pltpu.VMEM((_C,), jnp.float32),
          pltpu.VMEM((_C,), jnp.float32),
          pltpu.VMEM((_C,), jnp.int32),
          pltpu.VMEM((_C,), jnp.int32),
          pltpu.VMEM((_C,), jnp.int32),
          pltpu.VMEM((_C,), jnp.int32),
          pltpu.VMEM((_C,), jnp.float32),
          pltpu.VMEM((_C,), jnp.float32),
          pltpu.VMEM((_C,), jnp.float32),
          pltpu.VMEM((_C,), jnp.float32),
          pltpu.VMEM((4, 16), jnp.float32),
          pltpu.VMEM((3, 4, 16), jnp.float32),
          pltpu.VMEM((16,), jnp.float32),
          pltpu.VMEM((1, 16), jnp.float32),
          pltpu.SemaphoreType.DMA,
      ],
  )
  return f(pts, params, regp, aux, vox)


def kernel(point_cloud, auxiliary_data, voxel_data, predicted_planes):
  pts = jnp.transpose(point_cloud, (2, 0, 1))
  params = jnp.broadcast_to(
      predicted_planes.reshape(_P * _B, 4)[:, :, None], (_P * _B, 4, 16))
  regp = jnp.transpose(predicted_planes, (0, 2, 1))
  aux = auxiliary_data.reshape(_B * _S3 * 3)
  vox = voxel_data.reshape(_B * _S3)
  out = _sc_loss(pts, params, regp, aux, vox)
  return jnp.sum(out)
